# initial kernel scaffold (unmeasured)
import jax
import jax.numpy as jnp
from jax import lax
from jax.experimental import pallas as pl
from jax.experimental.pallas import tpu as pltpu

N_DEV = 32
LOG_N = 5
N_LAYERS = 3
N_STAGES = N_LAYERS * LOG_N


def kernel(x, Win0, Wout0, Win1, Wout1, Win2, Wout2):
    b, d_shard = x.shape
    h_dim = Win0.shape[1]

    def body(x_ref, win0_ref, wout0_ref, win1_ref, wout1_ref, win2_ref,
             wout2_ref, out_ref, send_ref, recv_ref, send_sems, recv_sems):
        my_pos = lax.axis_index("i")

        wins = [win0_ref, win1_ref, win2_ref]
        wouts = [wout0_ref, wout1_ref, wout2_ref]

        xv = x_ref[...].astype(jnp.bfloat16)
        for layer in range(N_LAYERS):
            w_in = wins[layer][...].astype(jnp.bfloat16)
            acc = jnp.dot(xv, w_in, preferred_element_type=jnp.float32)

            for k in range(LOG_N):
                s = layer * LOG_N + k
                partner = my_pos ^ (1 << k)
                send_ref[...] = acc
                rdma = pltpu.make_async_remote_copy(
                    src_ref=send_ref,
                    dst_ref=recv_ref.at[s],
                    send_sem=send_sems.at[s],
                    recv_sem=recv_sems.at[s],
                    device_id=(partner,),
                    device_id_type=pl.DeviceIdType.MESH,
                )
                rdma.start()
                rdma.wait()
                acc = acc + recv_ref[s]

            h = jnp.maximum(acc, 0.0).astype(jnp.bfloat16)
            w_out = wouts[layer][...].astype(jnp.bfloat16)
            y = jnp.dot(h, w_out, preferred_element_type=jnp.float32)
            if layer == N_LAYERS - 1:
                out_ref[...] = y
            else:
                xv = y.astype(jnp.bfloat16)

    return pl.pallas_call(
        body,
        out_shape=jax.ShapeDtypeStruct((b, d_shard), jnp.float32),
        in_specs=[pl.BlockSpec(memory_space=pltpu.VMEM)] * 7,
        out_specs=pl.BlockSpec(memory_space=pltpu.VMEM),
        scratch_shapes=[
            pltpu.VMEM((b, h_dim), jnp.float32),
            pltpu.VMEM((N_STAGES, b, h_dim), jnp.float32),
            pltpu.SemaphoreType.DMA((N_STAGES,)),
            pltpu.SemaphoreType.DMA((N_STAGES,)),
        ],
        compiler_params=pltpu.CompilerParams(collective_id=0),
    )(x, Win0, Wout0, Win1, Wout1, Win2, Wout2)


# baseline (device time: 77584 ns/iter reference)
import jax
import jax.numpy as jnp
from jax import lax
from jax.experimental import pallas as pl
from jax.experimental.pallas import tpu as pltpu

N_DEV = 32
LOG_N = 5
N_LAYERS = 3
N_STAGES = N_LAYERS * LOG_N


def kernel(x, Win0, Wout0, Win1, Wout1, Win2, Wout2):
    b, d_shard = x.shape
    h_dim = Win0.shape[1]

    def body(x_ref, win0_ref, wout0_ref, win1_ref, wout1_ref, win2_ref,
             wout2_ref, out_ref, send_ref, recv_ref, send_sems, recv_sems):
        my_pos = lax.axis_index("i")

        wins = [win0_ref, win1_ref, win2_ref]
        wouts = [wout0_ref, wout1_ref, wout2_ref]

        xv = x_ref[...].astype(jnp.bfloat16)
        for layer in range(N_LAYERS):
            w_in = wins[layer][...].astype(jnp.bfloat16)
            acc = jnp.dot(xv, w_in, preferred_element_type=jnp.float32)

            for k in range(LOG_N):
                s = layer * LOG_N + k
                partner = my_pos ^ (1 << k)
                send_ref[...] = acc
                rdma = pltpu.make_async_remote_copy(
                    src_ref=send_ref,
                    dst_ref=recv_ref.at[s],
                    send_sem=send_sems.at[s],
                    recv_sem=recv_sems.at[s],
                    device_id=(partner,),
                    device_id_type=pl.DeviceIdType.MESH,
                )
                rdma.start()
                rdma.wait()
                acc = acc + recv_ref[s]

            h = jnp.maximum(acc, 0.0).astype(jnp.bfloat16)
            w_out = wouts[layer][...].astype(jnp.bfloat16)
            y = jnp.dot(h, w_out, preferred_element_type=jnp.float32)
            if layer == N_LAYERS - 1:
                out_ref[...] = y
            else:
                xv = y.astype(jnp.bfloat16)

    return pl.pallas_call(
        body,
        out_shape=jax.ShapeDtypeStruct((b, d_shard), jnp.float32),
        in_specs=[pl.BlockSpec(memory_space=pltpu.VMEM)] * 7,
        out_specs=pl.BlockSpec(memory_space=pltpu.VMEM),
        scratch_shapes=[
            pltpu.VMEM((b, h_dim), jnp.float32),
            pltpu.VMEM((N_STAGES, b, h_dim), jnp.float32),
            pltpu.SemaphoreType.DMA((N_STAGES,)),
            pltpu.SemaphoreType.DMA((N_STAGES,)),
        ],
    )(x, Win0, Wout0, Win1, Wout1, Win2, Wout2)


# device time: 63095 ns/iter; 1.2296x vs baseline; 1.2296x over previous
import jax
import jax.numpy as jnp
from jax import lax
from jax.experimental import pallas as pl
from jax.experimental.pallas import tpu as pltpu

N_DEV = 32
LOG_N = 5
N_LAYERS = 3
N_STAGES = N_LAYERS * LOG_N


def kernel(x, Win0, Wout0, Win1, Wout1, Win2, Wout2):
    b, d_shard = x.shape
    h_dim = Win0.shape[1]

    def body(x_ref, win0_ref, wout0_ref, win1_ref, wout1_ref, win2_ref,
             wout2_ref, out_ref, send_ref, recv_ref, send_sems, recv_sems):
        my_pos = lax.axis_index("i")

        wins = [win0_ref, win1_ref, win2_ref]
        wouts = [wout0_ref, wout1_ref, wout2_ref]

        xv = x_ref[...].astype(jnp.bfloat16)
        for layer in range(N_LAYERS):
            w_in = wins[layer][...].astype(jnp.bfloat16)
            acc = jnp.dot(xv, w_in, preferred_element_type=jnp.float32)

            for k in range(LOG_N):
                s = layer * LOG_N + k
                partner = my_pos ^ (1 << k)
                send_ref[...] = acc.astype(jnp.bfloat16)
                rdma = pltpu.make_async_remote_copy(
                    src_ref=send_ref,
                    dst_ref=recv_ref.at[s],
                    send_sem=send_sems.at[s],
                    recv_sem=recv_sems.at[s],
                    device_id=(partner,),
                    device_id_type=pl.DeviceIdType.MESH,
                )
                rdma.start()
                rdma.wait()
                acc = acc + recv_ref[s].astype(jnp.float32)

            h = jnp.maximum(acc, 0.0).astype(jnp.bfloat16)
            w_out = wouts[layer][...].astype(jnp.bfloat16)
            y = jnp.dot(h, w_out, preferred_element_type=jnp.float32)
            if layer == N_LAYERS - 1:
                out_ref[...] = y
            else:
                xv = y.astype(jnp.bfloat16)

    return pl.pallas_call(
        body,
        out_shape=jax.ShapeDtypeStruct((b, d_shard), jnp.float32),
        in_specs=[pl.BlockSpec(memory_space=pltpu.VMEM)] * 7,
        out_specs=pl.BlockSpec(memory_space=pltpu.VMEM),
        scratch_shapes=[
            pltpu.VMEM((b, h_dim), jnp.bfloat16),
            pltpu.VMEM((N_STAGES, b, h_dim), jnp.bfloat16),
            pltpu.SemaphoreType.DMA((N_STAGES,)),
            pltpu.SemaphoreType.DMA((N_STAGES,)),
        ],
    )(x, Win0, Wout0, Win1, Wout1, Win2, Wout2)


# device time: 55822 ns/iter; 1.3898x vs baseline; 1.1303x over previous
import jax
import jax.numpy as jnp
from jax import lax
from jax.experimental import pallas as pl
from jax.experimental.pallas import tpu as pltpu

N_DEV = 32
N_LAYERS = 3
STAGES = ((4, 0), (4, 2), (2, 4))
SLOTS_PER_LAYER = sum(r - 1 for r, _ in STAGES)
N_SLOTS = N_LAYERS * SLOTS_PER_LAYER


def kernel(x, Win0, Wout0, Win1, Wout1, Win2, Wout2):
    b, d_shard = x.shape
    h_dim = Win0.shape[1]

    def body(x_ref, win0_ref, wout0_ref, win1_ref, wout1_ref, win2_ref,
             wout2_ref, out_ref, send_ref, recv_ref, send_sems, recv_sems):
        my_pos = lax.axis_index("i")

        wins = [win0_ref, win1_ref, win2_ref]
        wouts = [wout0_ref, wout1_ref, wout2_ref]

        xv = x_ref[...].astype(jnp.bfloat16)
        for layer in range(N_LAYERS):
            w_in = wins[layer][...].astype(jnp.bfloat16)
            acc = jnp.dot(xv, w_in, preferred_element_type=jnp.float32)

            slot = layer * SLOTS_PER_LAYER
            for radix, shift in STAGES:
                send_ref[...] = acc.astype(jnp.bfloat16)
                rdmas = []
                base = slot
                for j in range(1, radix):
                    partner = my_pos ^ (j << shift)
                    rdma = pltpu.make_async_remote_copy(
                        src_ref=send_ref,
                        dst_ref=recv_ref.at[slot],
                        send_sem=send_sems.at[slot],
                        recv_sem=recv_sems.at[slot],
                        device_id=(partner,),
                        device_id_type=pl.DeviceIdType.MESH,
                    )
                    rdma.start()
                    rdmas.append(rdma)
                    slot += 1
                for rdma in rdmas:
                    rdma.wait()
                for s in range(base, slot):
                    acc = acc + recv_ref[s].astype(jnp.float32)

            h = jnp.maximum(acc, 0.0).astype(jnp.bfloat16)
            w_out = wouts[layer][...].astype(jnp.bfloat16)
            y = jnp.dot(h, w_out, preferred_element_type=jnp.float32)
            if layer == N_LAYERS - 1:
                out_ref[...] = y
            else:
                xv = y.astype(jnp.bfloat16)

    return pl.pallas_call(
        body,
        out_shape=jax.ShapeDtypeStruct((b, d_shard), jnp.float32),
        in_specs=[pl.BlockSpec(memory_space=pltpu.VMEM)] * 7,
        out_specs=pl.BlockSpec(memory_space=pltpu.VMEM),
        scratch_shapes=[
            pltpu.VMEM((b, h_dim), jnp.bfloat16),
            pltpu.VMEM((N_SLOTS, b, h_dim), jnp.bfloat16),
            pltpu.SemaphoreType.DMA((N_SLOTS,)),
            pltpu.SemaphoreType.DMA((N_SLOTS,)),
        ],
    )(x, Win0, Wout0, Win1, Wout1, Win2, Wout2)


# device time: 51169 ns/iter; 1.5162x vs baseline; 1.0909x over previous
import jax
import jax.numpy as jnp
from jax import lax
from jax.experimental import pallas as pl
from jax.experimental.pallas import tpu as pltpu

N_DEV = 32
N_LAYERS = 3
STAGES = ((2, 0), (4, 1), (4, 3))
SLOTS_PER_LAYER = sum(r - 1 for r, _ in STAGES)
N_SLOTS = N_LAYERS * SLOTS_PER_LAYER


def kernel(x, Win0, Wout0, Win1, Wout1, Win2, Wout2):
    b, d_shard = x.shape
    h_dim = Win0.shape[1]

    def body(x_ref, win0_ref, wout0_ref, win1_ref, wout1_ref, win2_ref,
             wout2_ref, out_ref, send_ref, recv_ref, send_sems, recv_sems):
        my_pos = lax.axis_index("i")

        wins = [win0_ref, win1_ref, win2_ref]
        wouts = [wout0_ref, wout1_ref, wout2_ref]

        xv = x_ref[...].astype(jnp.bfloat16)
        for layer in range(N_LAYERS):
            w_in = wins[layer][...].astype(jnp.bfloat16)
            acc = jnp.dot(xv, w_in, preferred_element_type=jnp.float32)

            if layer == 0:
                barrier_sem = pltpu.get_barrier_semaphore()
                n_partners = 0
                for radix, shift in STAGES:
                    for j in range(1, radix):
                        pl.semaphore_signal(
                            barrier_sem,
                            inc=1,
                            device_id=(my_pos ^ (j << shift),),
                            device_id_type=pl.DeviceIdType.MESH,
                        )
                        n_partners += 1
                pl.semaphore_wait(barrier_sem, n_partners)

            slot = layer * SLOTS_PER_LAYER
            for radix, shift in STAGES:
                send_ref[...] = acc.astype(jnp.bfloat16)
                rdmas = []
                base = slot
                for j in range(1, radix):
                    partner = my_pos ^ (j << shift)
                    rdma = pltpu.make_async_remote_copy(
                        src_ref=send_ref,
                        dst_ref=recv_ref.at[slot],
                        send_sem=send_sems.at[slot],
                        recv_sem=recv_sems.at[slot],
                        device_id=(partner,),
                        device_id_type=pl.DeviceIdType.MESH,
                    )
                    rdma.start()
                    rdmas.append(rdma)
                    slot += 1
                for rdma in rdmas:
                    rdma.wait()
                for s in range(base, slot):
                    acc = acc + recv_ref[s].astype(jnp.float32)

            h = jnp.maximum(acc, 0.0).astype(jnp.bfloat16)
            w_out = wouts[layer][...].astype(jnp.bfloat16)
            y = jnp.dot(h, w_out, preferred_element_type=jnp.float32)
            if layer == N_LAYERS - 1:
                out_ref[...] = y
            else:
                xv = y.astype(jnp.bfloat16)

    return pl.pallas_call(
        body,
        out_shape=jax.ShapeDtypeStruct((b, d_shard), jnp.float32),
        in_specs=[pl.BlockSpec(memory_space=pltpu.VMEM)] * 7,
        out_specs=pl.BlockSpec(memory_space=pltpu.VMEM),
        scratch_shapes=[
            pltpu.VMEM((b, h_dim), jnp.bfloat16),
            pltpu.VMEM((N_SLOTS, b, h_dim), jnp.bfloat16),
            pltpu.SemaphoreType.DMA((N_SLOTS,)),
            pltpu.SemaphoreType.DMA((N_SLOTS,)),
        ],
        compiler_params=pltpu.CompilerParams(collective_id=0),
    )(x, Win0, Wout0, Win1, Wout1, Win2, Wout2)


# device time: 46862 ns/iter; 1.6556x vs baseline; 1.0919x over previous
import jax
import jax.numpy as jnp
from jax import lax
from jax.experimental import pallas as pl
from jax.experimental.pallas import tpu as pltpu

N_DEV = 32
N_LAYERS = 3
STAGES = ((4, 0), (2, 2), (4, 3))
SLOTS_PER_LAYER = sum(r - 1 for r, _ in STAGES)
N_SLOTS = max(1, N_LAYERS * SLOTS_PER_LAYER)


def kernel(x, Win0, Wout0, Win1, Wout1, Win2, Wout2):
    b, d_shard = x.shape
    h_dim = Win0.shape[1]

    def body(x_ref, win0_ref, wout0_ref, win1_ref, wout1_ref, win2_ref,
             wout2_ref, out_ref, send_ref, recv_ref, send_sems, recv_sems):
        my_pos = lax.axis_index("i")

        wins = [win0_ref, win1_ref, win2_ref]
        wouts = [wout0_ref, wout1_ref, wout2_ref]

        xv = x_ref[...].astype(jnp.bfloat16)
        for layer in range(N_LAYERS):
            with jax.named_scope(f"mm_in#l{layer}"):
                w_in = wins[layer][...].astype(jnp.bfloat16)
                acc = jnp.dot(xv, w_in, preferred_element_type=jnp.float32)

            if layer == 0 and STAGES:
                with jax.named_scope("barrier"):
                    barrier_sem = pltpu.get_barrier_semaphore()
                    n_partners = 0
                    for radix, shift in STAGES:
                        for j in range(1, radix):
                            pl.semaphore_signal(
                                barrier_sem,
                                inc=1,
                                device_id=(my_pos ^ (j << shift),),
                                device_id_type=pl.DeviceIdType.MESH,
                            )
                            n_partners += 1
                    pl.semaphore_wait(barrier_sem, n_partners)

            slot = layer * SLOTS_PER_LAYER
            for si, (radix, shift) in enumerate(STAGES):
                with jax.named_scope(f"store#l{layer}s{si}"):
                    send_ref[...] = acc.astype(jnp.bfloat16)
                rdmas = []
                base = slot
                with jax.named_scope(f"start#l{layer}s{si}"):
                    for j in range(1, radix):
                        partner = my_pos ^ (j << shift)
                        rdma = pltpu.make_async_remote_copy(
                            src_ref=send_ref,
                            dst_ref=recv_ref.at[slot],
                            send_sem=send_sems.at[slot],
                            recv_sem=recv_sems.at[slot],
                            device_id=(partner,),
                            device_id_type=pl.DeviceIdType.MESH,
                        )
                        rdma.start()
                        rdmas.append(rdma)
                        slot += 1
                with jax.named_scope(f"wait#l{layer}s{si}"):
                    for rdma in rdmas:
                        rdma.wait()
                with jax.named_scope(f"reduce#l{layer}s{si}"):
                    for s in range(base, slot):
                        acc = acc + recv_ref[s].astype(jnp.float32)

            with jax.named_scope(f"mm_out#l{layer}"):
                h = jnp.maximum(acc, 0.0).astype(jnp.bfloat16)
                w_out = wouts[layer][...].astype(jnp.bfloat16)
                y = jnp.dot(h, w_out, preferred_element_type=jnp.float32)
                if layer == N_LAYERS - 1:
                    out_ref[...] = y
                else:
                    xv = y.astype(jnp.bfloat16)

    return pl.pallas_call(
        body,
        out_shape=jax.ShapeDtypeStruct((b, d_shard), jnp.float32),
        in_specs=[pl.BlockSpec(memory_space=pltpu.VMEM)] * 7,
        out_specs=pl.BlockSpec(memory_space=pltpu.VMEM),
        scratch_shapes=[
            pltpu.VMEM((b, h_dim), jnp.bfloat16),
            pltpu.VMEM((N_SLOTS, b, h_dim), jnp.bfloat16),
            pltpu.SemaphoreType.DMA((N_SLOTS,)),
            pltpu.SemaphoreType.DMA((N_SLOTS,)),
        ],
        compiler_params=(
            pltpu.CompilerParams(collective_id=0)
            if STAGES
            else pltpu.CompilerParams()
        ),
    )(x, Win0, Wout0, Win1, Wout1, Win2, Wout2)
